# baseline (speedup 1.0000x reference)
import functools

import jax
import jax.numpy as jnp
from jax import lax
from jax.experimental import pallas as pl
from jax.experimental.pallas import tpu as pltpu
from jax.experimental.pallas import tpu_sc as plsc

N = 10000
D = 128
E = 320000

NC = 2
NS = 16
NW = NC * NS
CH = 80
EPT = 10240
NCH = EPT // CH
NBUF = 4
NGRP = NCH // NBUF
ROWS_PER_TILE = 624
ROWS_REM = N - NS * ROWS_PER_TILE

_mesh = plsc.VectorSubcoreMesh(core_axis_name="c", subcore_axis_name="s")


@functools.partial(
    pl.kernel,
    mesh=_mesh,
    out_type=jax.ShapeDtypeStruct((NC, N + 8), jnp.int32),
    scratch_types=(
        [pltpu.VMEM((CH,), jnp.int32) for _ in range(NBUF)]
        + [pltpu.VMEM((CH,), jnp.int32)]
        + [pltpu.VMEM_SHARED((N + 8,), jnp.int32)]
        + [pltpu.SemaphoreType.DMA for _ in range(NBUF)]
    ),
)
def _sc_counts(dstp_hbm, zeros_hbm, ones_hbm, out_hbm, *scr):
    dbuf = scr[:NBUF]
    ones_v = scr[NBUF]
    csh = scr[NBUF + 1]
    sems = scr[NBUF + 2:]
    c = lax.axis_index("c")
    s = lax.axis_index("s")
    wid = c * NS + s

    pltpu.sync_copy(ones_hbm, ones_v)

    @pl.when(s == 0)
    def _():
        pltpu.sync_copy(zeros_hbm, csh)

    plsc.subcore_barrier()

    base = wid * EPT
    for k in range(NBUF):
        pltpu.sync_copy(dstp_hbm.at[pl.ds(base + k * CH, CH)], dbuf[k])

    def body(g, carry):
        for k in range(NBUF):
            pltpu.async_copy(ones_v, csh.at[dbuf[k]], sems[k], add=True)
        for k in range(NBUF):
            pltpu.make_async_copy(ones_v, csh.at[dbuf[k]], sems[k]).wait()
            off = base + ((g + 1) * NBUF + k) * CH
            pltpu.sync_copy(dstp_hbm.at[pl.ds(off, CH)], dbuf[k])
        return carry

    lax.fori_loop(0, NGRP - 1, body, 0)
    for k in range(NBUF):
        pltpu.async_copy(ones_v, csh.at[dbuf[k]], sems[k], add=True)
    for k in range(NBUF):
        pltpu.make_async_copy(ones_v, csh.at[dbuf[k]], sems[k]).wait()

    plsc.subcore_barrier()

    @pl.when(s == 0)
    def _():
        pltpu.sync_copy(csh, out_hbm.at[c])


@functools.partial(
    pl.kernel,
    mesh=_mesh,
    out_type=jax.ShapeDtypeStruct((NC, N, D), jnp.float32),
    scratch_types=(
        [pltpu.VMEM((CH,), jnp.int32) for _ in range(2 * NBUF)]
        + [pltpu.VMEM((CH, D), jnp.float32) for _ in range(NBUF)]
        + [pltpu.VMEM_SHARED((N + 8, D), jnp.float32)]
        + [pltpu.SemaphoreType.DMA for _ in range(2 * NBUF)]
    ),
)
def _sc_scatter(y_hbm, srcp_hbm, dstp_hbm, zf_hbm, out_hbm, *scr):
    sbuf = scr[:NBUF]
    dbuf = scr[NBUF:2 * NBUF]
    rows = scr[2 * NBUF:3 * NBUF]
    aggsh = scr[3 * NBUF]
    gsem = scr[3 * NBUF + 1:3 * NBUF + 1 + NBUF]
    ssem = scr[3 * NBUF + 1 + NBUF:]
    c = lax.axis_index("c")
    s = lax.axis_index("s")
    wid = c * NS + s

    pltpu.sync_copy(
        zf_hbm.at[pl.ds(s * ROWS_PER_TILE, ROWS_PER_TILE)],
        aggsh.at[pl.ds(s * ROWS_PER_TILE, ROWS_PER_TILE)],
    )

    @pl.when(s == 0)
    def _():
        pltpu.sync_copy(
            zf_hbm.at[pl.ds(NS * ROWS_PER_TILE, ROWS_REM)],
            aggsh.at[pl.ds(NS * ROWS_PER_TILE, ROWS_REM)],
        )

    plsc.subcore_barrier()

    base = wid * EPT

    def body(i, carry):
        off = base + i * CH
        pltpu.sync_copy(srcp_hbm.at[pl.ds(off, CH)], sbuf[0])
        pltpu.async_copy(y_hbm.at[sbuf[0]], rows[0], gsem[0]).wait()
        pltpu.sync_copy(dstp_hbm.at[pl.ds(off, CH)], dbuf[0])
        pltpu.sync_copy(rows[0], aggsh.at[dbuf[0]], add=True)
        return carry

    lax.fori_loop(0, NCH, body, 0)

    plsc.subcore_barrier()

    pltpu.sync_copy(
        aggsh.at[pl.ds(s * ROWS_PER_TILE, ROWS_PER_TILE)],
        out_hbm.at[c, pl.ds(s * ROWS_PER_TILE, ROWS_PER_TILE)],
    )

    @pl.when(s == 0)
    def _():
        pltpu.sync_copy(
            aggsh.at[pl.ds(NS * ROWS_PER_TILE, ROWS_REM)],
            out_hbm.at[c, pl.ds(NS * ROWS_PER_TILE, ROWS_REM)],
        )


_BLK = 1000
_GRID = N // _BLK


def _pre_body(cnt_ref, x_ref, w_ref, dinv_ref, y_ref):
    cnt = cnt_ref[0] + cnt_ref[1] + 1
    dinv = lax.rsqrt(cnt.astype(jnp.float32))
    dinv_ref[...] = dinv
    y_ref[...] = jnp.dot(x_ref[...], w_ref[...],
                         preferred_element_type=jnp.float32) * dinv


_tc_pre = pl.pallas_call(
    _pre_body,
    grid=(_GRID,),
    in_specs=[
        pl.BlockSpec((NC, _BLK, 1), lambda i: (0, i, 0)),
        pl.BlockSpec((_BLK, D), lambda i: (i, 0)),
        pl.BlockSpec((D, D), lambda i: (0, 0)),
    ],
    out_specs=[
        pl.BlockSpec((_BLK, 1), lambda i: (i, 0)),
        pl.BlockSpec((_BLK, D), lambda i: (i, 0)),
    ],
    out_shape=[
        jax.ShapeDtypeStruct((N, 1), jnp.float32),
        jax.ShapeDtypeStruct((N, D), jnp.float32),
    ],
)


def _fuse_body(agg_ref, y_ref, dinv_ref, b_ref, h_ref, ps_ref, psq_ref,
               ps_acc, psq_acc):
    i = pl.program_id(0)
    a = agg_ref[0] + agg_ref[1] + y_ref[...]
    t = a * dinv_ref[...] + b_ref[...]
    h = jnp.maximum(t, 0.0)
    h_ref[...] = h
    s1 = jnp.sum(h, axis=0, keepdims=True)
    s2 = jnp.sum(h * h, axis=0, keepdims=True)

    @pl.when(i == 0)
    def _():
        ps_acc[...] = jnp.zeros_like(ps_acc)
        psq_acc[...] = jnp.zeros_like(psq_acc)

    ps_acc[...] += s1
    psq_acc[...] += s2

    @pl.when(i == _GRID - 1)
    def _():
        ps_ref[...] = ps_acc[...]
        psq_ref[...] = psq_acc[...]


_tc_fuse = pl.pallas_call(
    _fuse_body,
    grid=(_GRID,),
    in_specs=[
        pl.BlockSpec((NC, _BLK, D), lambda i: (0, i, 0)),
        pl.BlockSpec((_BLK, D), lambda i: (i, 0)),
        pl.BlockSpec((_BLK, 1), lambda i: (i, 0)),
        pl.BlockSpec((1, D), lambda i: (0, 0)),
    ],
    out_specs=[
        pl.BlockSpec((_BLK, D), lambda i: (i, 0)),
        pl.BlockSpec((1, D), lambda i: (0, 0)),
        pl.BlockSpec((1, D), lambda i: (0, 0)),
    ],
    out_shape=[
        jax.ShapeDtypeStruct((N, D), jnp.float32),
        jax.ShapeDtypeStruct((1, D), jnp.float32),
        jax.ShapeDtypeStruct((1, D), jnp.float32),
    ],
    scratch_shapes=[
        pltpu.VMEM((1, D), jnp.float32),
        pltpu.VMEM((1, D), jnp.float32),
    ],
)


def _bn_scale_shift(ps_ref, psq_ref, g_ref, be_ref):
    mean = ps_ref[0] / N
    ex2 = psq_ref[0] / N
    var = ex2 - mean * mean
    sc = g_ref[0] * lax.rsqrt(var + 1e-5)
    sh = be_ref[0] - mean * sc
    return sc, sh


def _next_body(h_ref, ps_ref, psq_ref, g_ref, be_ref, dinv_ref, w_ref, y_ref):
    sc, sh = _bn_scale_shift(ps_ref, psq_ref, g_ref, be_ref)
    z = h_ref[...] * sc[None, :] + sh[None, :]
    y_ref[...] = jnp.dot(z, w_ref[...],
                         preferred_element_type=jnp.float32) * dinv_ref[...]


_tc_next = pl.pallas_call(
    _next_body,
    grid=(_GRID,),
    in_specs=[
        pl.BlockSpec((_BLK, D), lambda i: (i, 0)),
        pl.BlockSpec((1, D), lambda i: (0, 0)),
        pl.BlockSpec((1, D), lambda i: (0, 0)),
        pl.BlockSpec((1, D), lambda i: (0, 0)),
        pl.BlockSpec((1, D), lambda i: (0, 0)),
        pl.BlockSpec((_BLK, 1), lambda i: (i, 0)),
        pl.BlockSpec((D, D), lambda i: (0, 0)),
    ],
    out_specs=pl.BlockSpec((_BLK, D), lambda i: (i, 0)),
    out_shape=jax.ShapeDtypeStruct((N, D), jnp.float32),
)


def _final_body(h_ref, ps_ref, psq_ref, g_ref, be_ref, out_ref):
    sc, sh = _bn_scale_shift(ps_ref, psq_ref, g_ref, be_ref)
    out_ref[...] = h_ref[...] * sc[None, :] + sh[None, :]


_tc_final = pl.pallas_call(
    _final_body,
    grid=(_GRID,),
    in_specs=[
        pl.BlockSpec((_BLK, D), lambda i: (i, 0)),
        pl.BlockSpec((1, D), lambda i: (0, 0)),
        pl.BlockSpec((1, D), lambda i: (0, 0)),
        pl.BlockSpec((1, D), lambda i: (0, 0)),
        pl.BlockSpec((1, D), lambda i: (0, 0)),
    ],
    out_specs=pl.BlockSpec((_BLK, D), lambda i: (i, 0)),
    out_shape=jax.ShapeDtypeStruct((N, D), jnp.float32),
)


def kernel(x, edge_index, W1, b1, gamma1, beta1, W2, b2, gamma2, beta2,
           W3, b3, gamma3, beta3):
    src = edge_index[0].astype(jnp.int32)
    dst = edge_index[1].astype(jnp.int32)

    ept_real = E // NW
    pad = EPT - ept_real
    srcp = jnp.pad(src.reshape(NW, ept_real), ((0, 0), (0, pad)),
                   constant_values=0).reshape(-1)
    dstp = jnp.pad(dst.reshape(NW, ept_real), ((0, 0), (0, pad)),
                   constant_values=N).reshape(-1)

    zeros_i = jnp.zeros((N + 8,), jnp.int32)
    zeros_f = jnp.zeros((N, D), jnp.float32)
    ones_i = jnp.ones((CH,), jnp.int32)

    counts = _sc_counts(dstp, zeros_i, ones_i)
    counts = counts[:, :N]
    dinv, y = _tc_pre(counts.reshape(NC, N, 1), x, W1)

    params = [
        (b1, gamma1, beta1, W2),
        (b2, gamma2, beta2, W3),
        (b3, gamma3, beta3, None),
    ]
    out = None
    for b, g, be, w_next in params:
        aggs = _sc_scatter(y, srcp, dstp, zeros_f)
        h, ps, psq = _tc_fuse(aggs, y, dinv, b.reshape(1, D))
        if w_next is not None:
            y = _tc_next(h, ps, psq, g.reshape(1, D), be.reshape(1, D),
                         dinv, w_next)
        else:
            out = _tc_final(h, ps, psq, g.reshape(1, D), be.reshape(1, D))
    return out


# speedup vs baseline: 2.8211x; 2.8211x over previous
import functools

import jax
import jax.numpy as jnp
from jax import lax
from jax.experimental import pallas as pl
from jax.experimental.pallas import tpu as pltpu
from jax.experimental.pallas import tpu_sc as plsc

N = 10000
D = 128
E = 320000

NC = 2
NS = 16
NW = NC * NS
CH = 80
EPT = 10240
NCH = EPT // CH
NBUF = 4
NPAD = 256
NGRP = NCH // NBUF
ROWS_PER_TILE = 624
ROWS_REM = N - NS * ROWS_PER_TILE

_mesh = plsc.VectorSubcoreMesh(core_axis_name="c", subcore_axis_name="s")


@functools.partial(
    pl.kernel,
    mesh=_mesh,
    out_type=jax.ShapeDtypeStruct((NC, N + NPAD), jnp.int32),
    scratch_types=(
        [pltpu.VMEM((CH,), jnp.int32) for _ in range(NBUF)]
        + [pltpu.VMEM((CH,), jnp.int32)]
        + [pltpu.VMEM_SHARED((N + NPAD,), jnp.int32)]
        + [pltpu.SemaphoreType.DMA for _ in range(NBUF)]
    ),
)
def _sc_counts(dstp_hbm, zeros_hbm, ones_hbm, out_hbm, *scr):
    dbuf = scr[:NBUF]
    ones_v = scr[NBUF]
    csh = scr[NBUF + 1]
    sems = scr[NBUF + 2:]
    c = lax.axis_index("c")
    s = lax.axis_index("s")
    wid = c * NS + s

    pltpu.sync_copy(ones_hbm, ones_v)

    @pl.when(s == 0)
    def _():
        pltpu.sync_copy(zeros_hbm, csh)

    plsc.subcore_barrier()

    base = wid * EPT
    for k in range(NBUF):
        pltpu.sync_copy(dstp_hbm.at[pl.ds(base + k * CH, CH)], dbuf[k])

    def body(g, carry):
        for k in range(NBUF):
            pltpu.async_copy(ones_v, csh.at[dbuf[k]], sems[k], add=True)
        for k in range(NBUF):
            pltpu.make_async_copy(ones_v, csh.at[dbuf[k]], sems[k]).wait()
            off = base + ((g + 1) * NBUF + k) * CH
            pltpu.sync_copy(dstp_hbm.at[pl.ds(off, CH)], dbuf[k])
        return carry

    lax.fori_loop(0, NGRP - 1, body, 0)
    for k in range(NBUF):
        pltpu.async_copy(ones_v, csh.at[dbuf[k]], sems[k], add=True)
    for k in range(NBUF):
        pltpu.make_async_copy(ones_v, csh.at[dbuf[k]], sems[k]).wait()

    plsc.subcore_barrier()

    @pl.when(s == 0)
    def _():
        pltpu.sync_copy(csh, out_hbm.at[c])


@functools.partial(
    pl.kernel,
    mesh=_mesh,
    out_type=jax.ShapeDtypeStruct((NC, N, D), jnp.float32),
    scratch_types=(
        [pltpu.VMEM((CH,), jnp.int32) for _ in range(2 * NBUF)]
        + [pltpu.VMEM((CH, D), jnp.float32) for _ in range(NBUF)]
        + [pltpu.VMEM_SHARED((N + NPAD, D), jnp.float32)]
        + [pltpu.SemaphoreType.DMA for _ in range(2 * NBUF)]
    ),
)
def _sc_scatter(y_hbm, srcp_hbm, dstp_hbm, zf_hbm, out_hbm, *scr):
    sbuf = scr[:NBUF]
    dbuf = scr[NBUF:2 * NBUF]
    rows = scr[2 * NBUF:3 * NBUF]
    aggsh = scr[3 * NBUF]
    gsem = scr[3 * NBUF + 1:3 * NBUF + 1 + NBUF]
    ssem = scr[3 * NBUF + 1 + NBUF:]
    c = lax.axis_index("c")
    s = lax.axis_index("s")
    wid = c * NS + s

    pltpu.sync_copy(
        zf_hbm.at[pl.ds(s * ROWS_PER_TILE, ROWS_PER_TILE)],
        aggsh.at[pl.ds(s * ROWS_PER_TILE, ROWS_PER_TILE)],
    )

    @pl.when(s == 0)
    def _():
        pltpu.sync_copy(
            zf_hbm.at[pl.ds(NS * ROWS_PER_TILE, ROWS_REM)],
            aggsh.at[pl.ds(NS * ROWS_PER_TILE, ROWS_REM)],
        )

    plsc.subcore_barrier()

    base = wid * EPT

    def fill(k, i):
        off = base + i * CH
        pltpu.sync_copy(srcp_hbm.at[pl.ds(off, CH)], sbuf[k])
        pltpu.sync_copy(dstp_hbm.at[pl.ds(off, CH)], dbuf[k])
        pltpu.async_copy(y_hbm.at[sbuf[k]], rows[k], gsem[k])

    for k in range(NBUF):
        fill(k, k)

    def body(g, carry):
        for k in range(NBUF):
            pltpu.make_async_copy(y_hbm.at[sbuf[k]], rows[k], gsem[k]).wait()
            pltpu.async_copy(rows[k], aggsh.at[dbuf[k]], ssem[k], add=True)
        for k in range(NBUF):
            pltpu.make_async_copy(rows[k], aggsh.at[dbuf[k]], ssem[k]).wait()
            fill(k, (g + 1) * NBUF + k)
        return carry

    lax.fori_loop(0, NGRP - 1, body, 0)
    for k in range(NBUF):
        pltpu.make_async_copy(y_hbm.at[sbuf[k]], rows[k], gsem[k]).wait()
        pltpu.async_copy(rows[k], aggsh.at[dbuf[k]], ssem[k], add=True)
    for k in range(NBUF):
        pltpu.make_async_copy(rows[k], aggsh.at[dbuf[k]], ssem[k]).wait()

    plsc.subcore_barrier()

    pltpu.sync_copy(
        aggsh.at[pl.ds(s * ROWS_PER_TILE, ROWS_PER_TILE)],
        out_hbm.at[c, pl.ds(s * ROWS_PER_TILE, ROWS_PER_TILE)],
    )

    @pl.when(s == 0)
    def _():
        pltpu.sync_copy(
            aggsh.at[pl.ds(NS * ROWS_PER_TILE, ROWS_REM)],
            out_hbm.at[c, pl.ds(NS * ROWS_PER_TILE, ROWS_REM)],
        )


_BLK = 1000
_GRID = N // _BLK


def _pre_body(cnt_ref, x_ref, w_ref, dinv_ref, y_ref):
    cnt = cnt_ref[0] + cnt_ref[1] + 1
    dinv = lax.rsqrt(cnt.astype(jnp.float32))
    dinv_ref[...] = dinv
    y_ref[...] = jnp.dot(x_ref[...], w_ref[...],
                         preferred_element_type=jnp.float32) * dinv


_tc_pre = pl.pallas_call(
    _pre_body,
    grid=(_GRID,),
    in_specs=[
        pl.BlockSpec((NC, _BLK, 1), lambda i: (0, i, 0)),
        pl.BlockSpec((_BLK, D), lambda i: (i, 0)),
        pl.BlockSpec((D, D), lambda i: (0, 0)),
    ],
    out_specs=[
        pl.BlockSpec((_BLK, 1), lambda i: (i, 0)),
        pl.BlockSpec((_BLK, D), lambda i: (i, 0)),
    ],
    out_shape=[
        jax.ShapeDtypeStruct((N, 1), jnp.float32),
        jax.ShapeDtypeStruct((N, D), jnp.float32),
    ],
)


def _fuse_body(agg_ref, y_ref, dinv_ref, b_ref, h_ref, ps_ref, psq_ref,
               ps_acc, psq_acc):
    i = pl.program_id(0)
    a = agg_ref[0] + agg_ref[1] + y_ref[...]
    t = a * dinv_ref[...] + b_ref[...]
    h = jnp.maximum(t, 0.0)
    h_ref[...] = h
    s1 = jnp.sum(h, axis=0, keepdims=True)
    s2 = jnp.sum(h * h, axis=0, keepdims=True)

    @pl.when(i == 0)
    def _():
        ps_acc[...] = jnp.zeros_like(ps_acc)
        psq_acc[...] = jnp.zeros_like(psq_acc)

    ps_acc[...] += s1
    psq_acc[...] += s2

    @pl.when(i == _GRID - 1)
    def _():
        ps_ref[...] = ps_acc[...]
        psq_ref[...] = psq_acc[...]


_tc_fuse = pl.pallas_call(
    _fuse_body,
    grid=(_GRID,),
    in_specs=[
        pl.BlockSpec((NC, _BLK, D), lambda i: (0, i, 0)),
        pl.BlockSpec((_BLK, D), lambda i: (i, 0)),
        pl.BlockSpec((_BLK, 1), lambda i: (i, 0)),
        pl.BlockSpec((1, D), lambda i: (0, 0)),
    ],
    out_specs=[
        pl.BlockSpec((_BLK, D), lambda i: (i, 0)),
        pl.BlockSpec((1, D), lambda i: (0, 0)),
        pl.BlockSpec((1, D), lambda i: (0, 0)),
    ],
    out_shape=[
        jax.ShapeDtypeStruct((N, D), jnp.float32),
        jax.ShapeDtypeStruct((1, D), jnp.float32),
        jax.ShapeDtypeStruct((1, D), jnp.float32),
    ],
    scratch_shapes=[
        pltpu.VMEM((1, D), jnp.float32),
        pltpu.VMEM((1, D), jnp.float32),
    ],
)


def _bn_scale_shift(ps_ref, psq_ref, g_ref, be_ref):
    mean = ps_ref[0] / N
    ex2 = psq_ref[0] / N
    var = ex2 - mean * mean
    sc = g_ref[0] * lax.rsqrt(var + 1e-5)
    sh = be_ref[0] - mean * sc
    return sc, sh


def _next_body(h_ref, ps_ref, psq_ref, g_ref, be_ref, dinv_ref, w_ref, y_ref):
    sc, sh = _bn_scale_shift(ps_ref, psq_ref, g_ref, be_ref)
    z = h_ref[...] * sc[None, :] + sh[None, :]
    y_ref[...] = jnp.dot(z, w_ref[...],
                         preferred_element_type=jnp.float32) * dinv_ref[...]


_tc_next = pl.pallas_call(
    _next_body,
    grid=(_GRID,),
    in_specs=[
        pl.BlockSpec((_BLK, D), lambda i: (i, 0)),
        pl.BlockSpec((1, D), lambda i: (0, 0)),
        pl.BlockSpec((1, D), lambda i: (0, 0)),
        pl.BlockSpec((1, D), lambda i: (0, 0)),
        pl.BlockSpec((1, D), lambda i: (0, 0)),
        pl.BlockSpec((_BLK, 1), lambda i: (i, 0)),
        pl.BlockSpec((D, D), lambda i: (0, 0)),
    ],
    out_specs=pl.BlockSpec((_BLK, D), lambda i: (i, 0)),
    out_shape=jax.ShapeDtypeStruct((N, D), jnp.float32),
)


def _final_body(h_ref, ps_ref, psq_ref, g_ref, be_ref, out_ref):
    sc, sh = _bn_scale_shift(ps_ref, psq_ref, g_ref, be_ref)
    out_ref[...] = h_ref[...] * sc[None, :] + sh[None, :]


_tc_final = pl.pallas_call(
    _final_body,
    grid=(_GRID,),
    in_specs=[
        pl.BlockSpec((_BLK, D), lambda i: (i, 0)),
        pl.BlockSpec((1, D), lambda i: (0, 0)),
        pl.BlockSpec((1, D), lambda i: (0, 0)),
        pl.BlockSpec((1, D), lambda i: (0, 0)),
        pl.BlockSpec((1, D), lambda i: (0, 0)),
    ],
    out_specs=pl.BlockSpec((_BLK, D), lambda i: (i, 0)),
    out_shape=jax.ShapeDtypeStruct((N, D), jnp.float32),
)


def kernel(x, edge_index, W1, b1, gamma1, beta1, W2, b2, gamma2, beta2,
           W3, b3, gamma3, beta3):
    src = edge_index[0].astype(jnp.int32)
    dst = edge_index[1].astype(jnp.int32)

    ept_real = E // NW
    pad = EPT - ept_real
    pj = jnp.arange(NW * pad, dtype=jnp.int32)
    pad_src = ((pj * 37) % N).reshape(NW, pad)
    pad_dst = (N + (pj % NPAD)).reshape(NW, pad)
    srcp = jnp.concatenate(
        [src.reshape(NW, ept_real), pad_src], axis=1).reshape(-1)
    dstp = jnp.concatenate(
        [dst.reshape(NW, ept_real), pad_dst], axis=1).reshape(-1)

    zeros_i = jnp.zeros((N + NPAD,), jnp.int32)
    zeros_f = jnp.zeros((N, D), jnp.float32)
    ones_i = jnp.ones((CH,), jnp.int32)

    counts = _sc_counts(dstp, zeros_i, ones_i)
    counts = counts[:, :N]
    dinv, y = _tc_pre(counts.reshape(NC, N, 1), x, W1)

    params = [
        (b1, gamma1, beta1, W2),
        (b2, gamma2, beta2, W3),
        (b3, gamma3, beta3, None),
    ]
    out = None
    for b, g, be, w_next in params:
        aggs = _sc_scatter(y, srcp, dstp, zeros_f)
        h, ps, psq = _tc_fuse(aggs, y, dinv, b.reshape(1, D))
        if w_next is not None:
            y = _tc_next(h, ps, psq, g.reshape(1, D), be.reshape(1, D),
                         dinv, w_next)
        else:
            out = _tc_final(h, ps, psq, g.reshape(1, D), be.reshape(1, D))
    return out


# speedup vs baseline: 2.9178x; 1.0343x over previous
import functools

import jax
import jax.numpy as jnp
from jax import lax
from jax.experimental import pallas as pl
from jax.experimental.pallas import tpu as pltpu
from jax.experimental.pallas import tpu_sc as plsc

N = 10000
D = 128
E = 320000

NC = 2
NS = 16
NW = NC * NS
CH = 80
EPT = 10240
NCH = EPT // CH
NBUF = 4
NPAD = 256
CHC = 128
NCHC = EPT // CHC
NGRPC = NCHC // NBUF
NGRP = NCH // NBUF
ROWS_PER_TILE = 624
ROWS_REM = N - NS * ROWS_PER_TILE

_mesh = plsc.VectorSubcoreMesh(core_axis_name="c", subcore_axis_name="s")


@functools.partial(
    pl.kernel,
    mesh=_mesh,
    out_type=jax.ShapeDtypeStruct((NC, N + NPAD), jnp.int32),
    scratch_types=(
        [pltpu.VMEM((CHC,), jnp.int32) for _ in range(NBUF)]
        + [pltpu.VMEM((CHC,), jnp.int32)]
        + [pltpu.VMEM_SHARED((N + NPAD,), jnp.int32)]
        + [pltpu.SemaphoreType.DMA for _ in range(NBUF)]
    ),
)
def _sc_counts(dstp_hbm, zeros_hbm, ones_hbm, out_hbm, *scr):
    dbuf = scr[:NBUF]
    ones_v = scr[NBUF]
    csh = scr[NBUF + 1]
    sems = scr[NBUF + 2:]
    c = lax.axis_index("c")
    s = lax.axis_index("s")
    wid = c * NS + s

    pltpu.sync_copy(ones_hbm, ones_v)

    @pl.when(s == 0)
    def _():
        pltpu.sync_copy(zeros_hbm, csh)

    plsc.subcore_barrier()

    base = wid * EPT
    for k in range(NBUF):
        pltpu.sync_copy(dstp_hbm.at[pl.ds(base + k * CHC, CHC)], dbuf[k])

    def body(g, carry):
        for k in range(NBUF):
            pltpu.async_copy(ones_v, csh.at[dbuf[k]], sems[k], add=True)
        for k in range(NBUF):
            pltpu.make_async_copy(ones_v, csh.at[dbuf[k]], sems[k]).wait()
            off = base + ((g + 1) * NBUF + k) * CHC
            pltpu.sync_copy(dstp_hbm.at[pl.ds(off, CHC)], dbuf[k])
        return carry

    lax.fori_loop(0, NGRPC - 1, body, 0)
    for k in range(NBUF):
        pltpu.async_copy(ones_v, csh.at[dbuf[k]], sems[k], add=True)
    for k in range(NBUF):
        pltpu.make_async_copy(ones_v, csh.at[dbuf[k]], sems[k]).wait()

    plsc.subcore_barrier()

    @pl.when(s == 0)
    def _():
        pltpu.sync_copy(csh, out_hbm.at[c])


@functools.partial(
    pl.kernel,
    mesh=_mesh,
    out_type=jax.ShapeDtypeStruct((NC, N, D), jnp.float32),
    scratch_types=(
        [pltpu.VMEM((CH,), jnp.int32) for _ in range(2 * NBUF)]
        + [pltpu.VMEM((CH, D), jnp.float32) for _ in range(NBUF)]
        + [pltpu.VMEM_SHARED((N + NPAD, D), jnp.float32)]
        + [pltpu.SemaphoreType.DMA for _ in range(2 * NBUF)]
    ),
)
def _sc_scatter(y_hbm, srcp_hbm, dstp_hbm, zf_hbm, out_hbm, *scr):
    sbuf = scr[:NBUF]
    dbuf = scr[NBUF:2 * NBUF]
    rows = scr[2 * NBUF:3 * NBUF]
    aggsh = scr[3 * NBUF]
    gsem = scr[3 * NBUF + 1:3 * NBUF + 1 + NBUF]
    ssem = scr[3 * NBUF + 1 + NBUF:]
    c = lax.axis_index("c")
    s = lax.axis_index("s")
    wid = c * NS + s

    pltpu.sync_copy(
        zf_hbm.at[pl.ds(s * ROWS_PER_TILE, ROWS_PER_TILE)],
        aggsh.at[pl.ds(s * ROWS_PER_TILE, ROWS_PER_TILE)],
    )

    @pl.when(s == 0)
    def _():
        pltpu.sync_copy(
            zf_hbm.at[pl.ds(NS * ROWS_PER_TILE, ROWS_REM)],
            aggsh.at[pl.ds(NS * ROWS_PER_TILE, ROWS_REM)],
        )

    plsc.subcore_barrier()

    base = wid * EPT

    def fill(k, i):
        off = base + i * CH
        pltpu.sync_copy(srcp_hbm.at[pl.ds(off, CH)], sbuf[k])
        pltpu.sync_copy(dstp_hbm.at[pl.ds(off, CH)], dbuf[k])
        pltpu.async_copy(y_hbm.at[sbuf[k]], rows[k], gsem[k])

    for k in range(NBUF):
        fill(k, k)

    def body(g, carry):
        for k in range(NBUF):
            pltpu.make_async_copy(y_hbm.at[sbuf[k]], rows[k], gsem[k]).wait()
            pltpu.async_copy(rows[k], aggsh.at[dbuf[k]], ssem[k], add=True)
        for k in range(NBUF):
            pltpu.make_async_copy(rows[k], aggsh.at[dbuf[k]], ssem[k]).wait()
            fill(k, (g + 1) * NBUF + k)
        return carry

    lax.fori_loop(0, NGRP - 1, body, 0)
    for k in range(NBUF):
        pltpu.make_async_copy(y_hbm.at[sbuf[k]], rows[k], gsem[k]).wait()
        pltpu.async_copy(rows[k], aggsh.at[dbuf[k]], ssem[k], add=True)
    for k in range(NBUF):
        pltpu.make_async_copy(rows[k], aggsh.at[dbuf[k]], ssem[k]).wait()

    plsc.subcore_barrier()

    pltpu.sync_copy(
        aggsh.at[pl.ds(s * ROWS_PER_TILE, ROWS_PER_TILE)],
        out_hbm.at[c, pl.ds(s * ROWS_PER_TILE, ROWS_PER_TILE)],
    )

    @pl.when(s == 0)
    def _():
        pltpu.sync_copy(
            aggsh.at[pl.ds(NS * ROWS_PER_TILE, ROWS_REM)],
            out_hbm.at[c, pl.ds(NS * ROWS_PER_TILE, ROWS_REM)],
        )


_BLK = 1000
_GRID = N // _BLK


def _pre_body(cnt_ref, x_ref, w_ref, dinv_ref, y_ref):
    cnt = cnt_ref[0] + cnt_ref[1] + 1
    dinv = lax.rsqrt(cnt.astype(jnp.float32))
    dinv_ref[...] = dinv
    y_ref[...] = jnp.dot(x_ref[...], w_ref[...],
                         preferred_element_type=jnp.float32) * dinv


_tc_pre = pl.pallas_call(
    _pre_body,
    grid=(_GRID,),
    in_specs=[
        pl.BlockSpec((NC, _BLK, 1), lambda i: (0, i, 0)),
        pl.BlockSpec((_BLK, D), lambda i: (i, 0)),
        pl.BlockSpec((D, D), lambda i: (0, 0)),
    ],
    out_specs=[
        pl.BlockSpec((_BLK, 1), lambda i: (i, 0)),
        pl.BlockSpec((_BLK, D), lambda i: (i, 0)),
    ],
    out_shape=[
        jax.ShapeDtypeStruct((N, 1), jnp.float32),
        jax.ShapeDtypeStruct((N, D), jnp.float32),
    ],
)


def _mk_layer(last):
    def body(agg_ref, y_ref, dinv_ref, b_ref, g_ref, be_ref, w_ref,
             out_ref, h_scr, ps_acc, psq_acc):
        p = pl.program_id(0)
        i = pl.program_id(1)

        @pl.when(p == 0)
        def _():
            a = agg_ref[0] + agg_ref[1] + y_ref[...]
            h = jnp.maximum(a * dinv_ref[...] + b_ref[...], 0.0)
            h_scr[pl.ds(i * _BLK, _BLK), :] = h
            s1 = jnp.sum(h, axis=0, keepdims=True)
            s2 = jnp.sum(h * h, axis=0, keepdims=True)

            @pl.when(i == 0)
            def _():
                ps_acc[...] = jnp.zeros_like(ps_acc)
                psq_acc[...] = jnp.zeros_like(psq_acc)

            ps_acc[...] += s1
            psq_acc[...] += s2

        @pl.when(p == 1)
        def _():
            mean = ps_acc[0] / N
            var = psq_acc[0] / N - mean * mean
            sc = g_ref[0] * lax.rsqrt(var + 1e-5)
            sh = be_ref[0] - mean * sc
            h = h_scr[pl.ds(i * _BLK, _BLK), :]
            z = h * sc[None, :] + sh[None, :]
            if last:
                out_ref[...] = z
            else:
                out_ref[...] = jnp.dot(
                    z, w_ref[...],
                    preferred_element_type=jnp.float32) * dinv_ref[...]

    return pl.pallas_call(
        body,
        grid=(2, _GRID),
        in_specs=[
            pl.BlockSpec((NC, _BLK, D), lambda p, i: (0, i * (1 - p), 0)),
            pl.BlockSpec((_BLK, D), lambda p, i: (i * (1 - p), 0)),
            pl.BlockSpec((_BLK, 1), lambda p, i: (i, 0)),
            pl.BlockSpec((1, D), lambda p, i: (0, 0)),
            pl.BlockSpec((1, D), lambda p, i: (0, 0)),
            pl.BlockSpec((1, D), lambda p, i: (0, 0)),
            pl.BlockSpec((D, D), lambda p, i: (0, 0)),
        ],
        out_specs=pl.BlockSpec((_BLK, D), lambda p, i: (i, 0)),
        out_shape=jax.ShapeDtypeStruct((N, D), jnp.float32),
        scratch_shapes=[
            pltpu.VMEM((N, D), jnp.float32),
            pltpu.VMEM((1, D), jnp.float32),
            pltpu.VMEM((1, D), jnp.float32),
        ],
    )


_tc_layer = _mk_layer(last=False)
_tc_last = _mk_layer(last=True)


def kernel(x, edge_index, W1, b1, gamma1, beta1, W2, b2, gamma2, beta2,
           W3, b3, gamma3, beta3):
    src = edge_index[0].astype(jnp.int32)
    dst = edge_index[1].astype(jnp.int32)

    ept_real = E // NW
    pad = EPT - ept_real
    pj = jnp.arange(NW * pad, dtype=jnp.int32)
    pad_src = ((pj * 37) % N).reshape(NW, pad)
    pad_dst = (N + (pj % NPAD)).reshape(NW, pad)
    srcp = jnp.concatenate(
        [src.reshape(NW, ept_real), pad_src], axis=1).reshape(-1)
    dstp = jnp.concatenate(
        [dst.reshape(NW, ept_real), pad_dst], axis=1).reshape(-1)

    zeros_i = jnp.zeros((N + NPAD,), jnp.int32)
    zeros_f = jnp.zeros((N, D), jnp.float32)
    ones_i = jnp.ones((CHC,), jnp.int32)

    counts = _sc_counts(dstp, zeros_i, ones_i)
    counts = counts[:, :N]
    dinv, y = _tc_pre(counts.reshape(NC, N, 1), x, W1)

    params = [
        (b1, gamma1, beta1, W2),
        (b2, gamma2, beta2, W3),
        (b3, gamma3, beta3, None),
    ]
    out = None
    for b, g, be, w_next in params:
        aggs = _sc_scatter(y, srcp, dstp, zeros_f)
        if w_next is not None:
            y = _tc_layer(aggs, y, dinv, b.reshape(1, D), g.reshape(1, D),
                          be.reshape(1, D), w_next)
        else:
            out = _tc_last(aggs, y, dinv, b.reshape(1, D), g.reshape(1, D),
                           be.reshape(1, D), W1)
    return out
